# baseline (speedup 1.0000x reference)
import functools

import jax
import jax.numpy as jnp
import numpy as np
from jax import lax
from jax.experimental import pallas as pl
from jax.experimental.pallas import tpu as pltpu
from jax.experimental.pallas import tpu_sc as plsc

B = 4
C = 3
HW = 224
P = 16
NP = (HW // P) ** 2
D_VIS = 768
D_MODEL = 1024
N_HEADS = 16
HD = D_MODEL // N_HEADS
VOCAB = 32000
PROMPT_LEN = 3
T = 1 + NP + 1 + PROMPT_LEN
TP = 208
CPP = C * P * P
PPC = P * P

VT = 3200
NV = VOCAB // VT

_PREC = jax.lax.Precision.DEFAULT


def _ln(x):
    mu = jnp.mean(x, axis=-1, keepdims=True)
    var = jnp.mean((x - mu) ** 2, axis=-1, keepdims=True)
    return (x - mu) * jax.lax.rsqrt(var + 1e-5)


def _sc_gather(table, idx):
    mesh = plsc.VectorSubcoreMesh(core_axis_name="c", subcore_axis_name="s")

    @functools.partial(
        pl.kernel, mesh=mesh,
        out_type=jax.ShapeDtypeStruct((8, D_MODEL), jnp.float32),
        scratch_types=[
            pltpu.VMEM((8,), jnp.int32),
            pltpu.VMEM((8, D_MODEL), jnp.float32),
            pltpu.SemaphoreType.DMA,
        ],
    )
    def k(table_hbm, idx_hbm, out_hbm, idx_v, rows_v, sem):
        wid = lax.axis_index("s") * 2 + lax.axis_index("c")

        @pl.when(wid == 0)
        def _():
            pltpu.sync_copy(idx_hbm, idx_v)
            pltpu.async_copy(table_hbm.at[idx_v], rows_v, sem).wait()
            pltpu.sync_copy(rows_v, out_hbm)

    return k(table, idx)


def _block_body(p_ref, emb_ref, wp_ref, wt_ref, wqkv_ref, wo_ref,
                w1_ref, w2_ref, out_ref):
    p = p_ref[0].astype(jnp.float32)
    acc = jnp.zeros((NP, D_VIS), jnp.float32)
    for c in range(C):
        pc = p[:, c * PPC:(c + 1) * PPC]
        mu = jnp.mean(pc)
        var = jnp.mean((pc - mu) ** 2)
        inv = 1.0 / (jnp.sqrt(var) + 1e-6)
        pcn = (pc - mu) * inv
        acc = acc + jax.lax.dot(pcn, wp_ref[c * PPC:(c + 1) * PPC, :],
                                precision=_PREC)
    vis = jax.lax.dot(acc, wt_ref[...], precision=_PREC)

    emb = emb_ref[...]
    x = jnp.concatenate(
        [emb[0:1], vis, emb[1:2], emb[2:2 + PROMPT_LEN],
         jnp.zeros((TP - T, D_MODEL), jnp.float32)], axis=0)

    h = _ln(x)
    qkv = jax.lax.dot(h, wqkv_ref[...], precision=_PREC)
    rows = jax.lax.broadcasted_iota(jnp.int32, (TP, TP), 0)
    cols = jax.lax.broadcasted_iota(jnp.int32, (TP, TP), 1)
    causal = rows >= cols
    outs = []
    for hh in range(N_HEADS):
        q = qkv[:, hh * HD:(hh + 1) * HD]
        k = qkv[:, D_MODEL + hh * HD:D_MODEL + (hh + 1) * HD]
        v = qkv[:, 2 * D_MODEL + hh * HD:2 * D_MODEL + (hh + 1) * HD]
        att = jax.lax.dot_general(q, k, (((1,), (1,)), ((), ())),
                                  precision=_PREC) * (1.0 / np.sqrt(HD))
        att = jnp.where(causal, att, jnp.float32(-1e30))
        att = jax.nn.softmax(att, axis=-1)
        outs.append(jax.lax.dot(att, v, precision=_PREC))
    o = jnp.concatenate(outs, axis=1)
    x = x + jax.lax.dot(o, wo_ref[...], precision=_PREC)

    h2 = _ln(x)
    g = jax.nn.gelu(jax.lax.dot(h2, w1_ref[...], precision=_PREC))
    xf = x + jax.lax.dot(g, w2_ref[...], precision=_PREC)
    out_ref[...] = xf.astype(jnp.bfloat16)


def _block(patches, emb, W_patch, W_tok, Wqkv, Wo, W1, W2):
    return pl.pallas_call(
        _block_body,
        grid=(B,),
        in_specs=[
            pl.BlockSpec((1, NP, CPP), lambda b: (b, 0, 0)),
            pl.BlockSpec((8, D_MODEL), lambda b: (0, 0)),
            pl.BlockSpec((CPP, D_VIS), lambda b: (0, 0)),
            pl.BlockSpec((D_VIS, D_MODEL), lambda b: (0, 0)),
            pl.BlockSpec((D_MODEL, 3 * D_MODEL), lambda b: (0, 0)),
            pl.BlockSpec((D_MODEL, D_MODEL), lambda b: (0, 0)),
            pl.BlockSpec((D_MODEL, 4 * D_MODEL), lambda b: (0, 0)),
            pl.BlockSpec((4 * D_MODEL, D_MODEL), lambda b: (0, 0)),
        ],
        out_specs=pl.BlockSpec((TP, D_MODEL), lambda b: (b, 0)),
        out_shape=jax.ShapeDtypeStruct((B * TP, D_MODEL), jnp.bfloat16),
        compiler_params=pltpu.CompilerParams(
            vmem_limit_bytes=100 * 1024 * 1024),
    )(patches, emb, W_patch, W_tok, Wqkv, Wo, W1, W2)


def _head_body(labels_ref, h_ref, w_ref, logits_ref, loss_ref,
               m_sc, s_sc, lab_sc):
    v = pl.program_id(0)
    h = h_ref[...]
    logits = jax.lax.dot_general(
        h, w_ref[...].astype(jnp.bfloat16), (((1,), (0,)), ((), ())),
        preferred_element_type=jnp.float32)
    lane = jax.lax.broadcasted_iota(jnp.int32, (VT,), 0)
    for b in range(B):
        logits_ref[b] = logits[b * TP:b * TP + T]
        last = logits[b * TP + T - 1, :]
        tmax = jnp.max(last)

        lab = labels_ref[b]
        local = lab - v * VT
        in_tile = (local >= 0) & (local < VT)
        contrib = jnp.sum(jnp.where(lane == local, last, 0.0))
        contrib = jnp.where(in_tile, contrib, 0.0)

        @pl.when(v == 0)
        def _init():
            m_sc[b] = tmax
            s_sc[b] = jnp.sum(jnp.exp(last - tmax))
            lab_sc[b] = contrib

        @pl.when(v > 0)
        def _update():
            m_old = m_sc[b]
            m_new = jnp.maximum(m_old, tmax)
            m_sc[b] = m_new
            s_sc[b] = (s_sc[b] * jnp.exp(m_old - m_new)
                       + jnp.sum(jnp.exp(last - m_new)))
            lab_sc[b] = lab_sc[b] + contrib

    @pl.when(v == NV - 1)
    def _finish():
        acc = jnp.float32(0.0)
        for bb in range(B):
            acc = acc + (jnp.log(s_sc[bb]) + m_sc[bb] - lab_sc[bb])
        loss_ref[0, 0] = acc / B


def _head(labels, h, W_head):
    return pl.pallas_call(
        _head_body,
        grid=(NV,),
        in_specs=[
            pl.BlockSpec(memory_space=pltpu.SMEM),
            pl.BlockSpec((B * TP, D_MODEL), lambda v: (0, 0)),
            pl.BlockSpec((D_MODEL, VT), lambda v: (0, v)),
        ],
        out_specs=[
            pl.BlockSpec((B, T, VT), lambda v: (0, 0, v)),
            pl.BlockSpec(memory_space=pltpu.SMEM),
        ],
        out_shape=[
            jax.ShapeDtypeStruct((B, T, VOCAB), jnp.float32),
            jax.ShapeDtypeStruct((1, 1), jnp.float32),
        ],
        scratch_shapes=[
            pltpu.SMEM((B,), jnp.float32),
            pltpu.SMEM((B,), jnp.float32),
            pltpu.SMEM((B,), jnp.float32),
        ],
        compiler_params=pltpu.CompilerParams(
            vmem_limit_bytes=120 * 1024 * 1024),
    )(labels, h, W_head)


def kernel(image, labels, W_patch, W_tok, embed_table, Wqkv, Wo, W1, W2,
           W_head, prompt_ids):
    img = image.reshape(B, C, HW // P, P, HW // P, P)
    patches = img.transpose(0, 2, 4, 1, 3, 5).reshape(B, NP, CPP)
    patches = patches.astype(jnp.bfloat16)

    idx = jnp.concatenate([jnp.array([VOCAB, VOCAB + 1], jnp.int32),
                           prompt_ids.astype(jnp.int32),
                           jnp.zeros((3,), jnp.int32)])
    emb = _sc_gather(embed_table, idx)

    x = _block(patches, emb, W_patch, W_tok, Wqkv, Wo, W1, W2)

    logits, loss = _head(labels.astype(jnp.int32), x, W_head)
    return logits, loss.reshape(())


# speedup vs baseline: 1.0483x; 1.0483x over previous
import functools

import jax
import jax.numpy as jnp
import numpy as np
from jax.experimental import pallas as pl
from jax.experimental.pallas import tpu as pltpu

B = 4
C = 3
HW = 224
P = 16
NP = (HW // P) ** 2
D_VIS = 768
D_MODEL = 1024
N_HEADS = 16
HD = D_MODEL // N_HEADS
VOCAB = 32000
PROMPT_LEN = 3
T = 1 + NP + 1 + PROMPT_LEN
TP = 208
CPP = C * P * P
PPC = P * P

VT = 3200
NV = VOCAB // VT

_PREC = jax.lax.Precision.DEFAULT


def _ln(x):
    mu = jnp.mean(x, axis=-1, keepdims=True)
    var = jnp.mean((x - mu) ** 2, axis=-1, keepdims=True)
    return (x - mu) * jax.lax.rsqrt(var + 1e-5)


def _block_body(p_ref, emb_ref, wp_ref, wt_ref, wqkv_ref, wo_ref,
                w1_ref, w2_ref, out_ref):
    p = p_ref[0].astype(jnp.float32)
    acc = jnp.zeros((NP, D_VIS), jnp.float32)
    for c in range(C):
        pc = p[:, c * PPC:(c + 1) * PPC]
        mu = jnp.mean(pc)
        var = jnp.mean((pc - mu) ** 2)
        inv = 1.0 / (jnp.sqrt(var) + 1e-6)
        pcn = (pc - mu) * inv
        acc = acc + jax.lax.dot(pcn, wp_ref[c * PPC:(c + 1) * PPC, :],
                                precision=_PREC)
    vis = jax.lax.dot(acc, wt_ref[...], precision=_PREC)

    emb = emb_ref[...]
    x = jnp.concatenate(
        [emb[0:1], vis, emb[1:2], emb[2:2 + PROMPT_LEN],
         jnp.zeros((TP - T, D_MODEL), jnp.float32)], axis=0)

    h = _ln(x)
    qkv = jax.lax.dot(h, wqkv_ref[...], precision=_PREC)
    rows = jax.lax.broadcasted_iota(jnp.int32, (TP, TP), 0)
    cols = jax.lax.broadcasted_iota(jnp.int32, (TP, TP), 1)
    causal = rows >= cols
    outs = []
    for hh in range(N_HEADS):
        q = qkv[:, hh * HD:(hh + 1) * HD]
        k = qkv[:, D_MODEL + hh * HD:D_MODEL + (hh + 1) * HD]
        v = qkv[:, 2 * D_MODEL + hh * HD:2 * D_MODEL + (hh + 1) * HD]
        att = jax.lax.dot_general(q, k, (((1,), (1,)), ((), ())),
                                  precision=_PREC) * (1.0 / np.sqrt(HD))
        att = jnp.where(causal, att, jnp.float32(-1e30))
        att = jax.nn.softmax(att, axis=-1)
        outs.append(jax.lax.dot(att, v, precision=_PREC))
    o = jnp.concatenate(outs, axis=1)
    x = x + jax.lax.dot(o, wo_ref[...], precision=_PREC)

    h2 = _ln(x)
    g = jax.nn.gelu(jax.lax.dot(h2, w1_ref[...], precision=_PREC))
    xf = x + jax.lax.dot(g, w2_ref[...], precision=_PREC)
    out_ref[...] = xf.astype(jnp.bfloat16)


def _block(patches, emb, W_patch, W_tok, Wqkv, Wo, W1, W2):
    return pl.pallas_call(
        _block_body,
        grid=(B,),
        in_specs=[
            pl.BlockSpec((1, NP, CPP), lambda b: (b, 0, 0)),
            pl.BlockSpec((8, D_MODEL), lambda b: (0, 0)),
            pl.BlockSpec((CPP, D_VIS), lambda b: (0, 0)),
            pl.BlockSpec((D_VIS, D_MODEL), lambda b: (0, 0)),
            pl.BlockSpec((D_MODEL, 3 * D_MODEL), lambda b: (0, 0)),
            pl.BlockSpec((D_MODEL, D_MODEL), lambda b: (0, 0)),
            pl.BlockSpec((D_MODEL, 4 * D_MODEL), lambda b: (0, 0)),
            pl.BlockSpec((4 * D_MODEL, D_MODEL), lambda b: (0, 0)),
        ],
        out_specs=pl.BlockSpec((TP, D_MODEL), lambda b: (b, 0)),
        out_shape=jax.ShapeDtypeStruct((B * TP, D_MODEL), jnp.bfloat16),
        compiler_params=pltpu.CompilerParams(
            vmem_limit_bytes=100 * 1024 * 1024),
    )(patches, emb, W_patch, W_tok, Wqkv, Wo, W1, W2)


def _head_body(labels_ref, h_ref, w_ref, logits_ref, loss_ref,
               m_sc, s_sc, lab_sc):
    v = pl.program_id(0)
    h = h_ref[...]
    logits = jax.lax.dot_general(
        h, w_ref[...].astype(jnp.bfloat16), (((1,), (0,)), ((), ())),
        preferred_element_type=jnp.float32)
    lane = jax.lax.broadcasted_iota(jnp.int32, (VT,), 0)
    for b in range(B):
        logits_ref[b] = logits[b * TP:b * TP + T]
        last = logits[b * TP + T - 1, :]
        tmax = jnp.max(last)

        lab = labels_ref[b]
        local = lab - v * VT
        in_tile = (local >= 0) & (local < VT)
        contrib = jnp.sum(jnp.where(lane == local, last, 0.0))
        contrib = jnp.where(in_tile, contrib, 0.0)

        @pl.when(v == 0)
        def _init():
            m_sc[b] = tmax
            s_sc[b] = jnp.sum(jnp.exp(last - tmax))
            lab_sc[b] = contrib

        @pl.when(v > 0)
        def _update():
            m_old = m_sc[b]
            m_new = jnp.maximum(m_old, tmax)
            m_sc[b] = m_new
            s_sc[b] = (s_sc[b] * jnp.exp(m_old - m_new)
                       + jnp.sum(jnp.exp(last - m_new)))
            lab_sc[b] = lab_sc[b] + contrib

    @pl.when(v == NV - 1)
    def _finish():
        acc = jnp.float32(0.0)
        for bb in range(B):
            acc = acc + (jnp.log(s_sc[bb]) + m_sc[bb] - lab_sc[bb])
        loss_ref[0, 0] = acc / B


def _head(labels, h, W_head):
    return pl.pallas_call(
        _head_body,
        grid=(NV,),
        in_specs=[
            pl.BlockSpec(memory_space=pltpu.SMEM),
            pl.BlockSpec((B * TP, D_MODEL), lambda v: (0, 0)),
            pl.BlockSpec((D_MODEL, VT), lambda v: (0, v)),
        ],
        out_specs=[
            pl.BlockSpec((B, T, VT), lambda v: (0, 0, v)),
            pl.BlockSpec(memory_space=pltpu.SMEM),
        ],
        out_shape=[
            jax.ShapeDtypeStruct((B, T, VOCAB), jnp.float32),
            jax.ShapeDtypeStruct((1, 1), jnp.float32),
        ],
        scratch_shapes=[
            pltpu.SMEM((B,), jnp.float32),
            pltpu.SMEM((B,), jnp.float32),
            pltpu.SMEM((B,), jnp.float32),
        ],
        compiler_params=pltpu.CompilerParams(
            vmem_limit_bytes=120 * 1024 * 1024),
    )(labels, h, W_head)


def kernel(image, labels, W_patch, W_tok, embed_table, Wqkv, Wo, W1, W2,
           W_head, prompt_ids):
    img = image.reshape(B, C, HW // P, P, HW // P, P)
    patches = img.transpose(0, 2, 4, 1, 3, 5).reshape(B, NP, CPP)
    patches = patches.astype(jnp.bfloat16)

    idx = jnp.concatenate([jnp.array([VOCAB, VOCAB + 1], jnp.int32),
                           prompt_ids.astype(jnp.int32),
                           jnp.zeros((3,), jnp.int32)])
    emb = jnp.take(embed_table, idx, axis=0)

    x = _block(patches, emb, W_patch, W_tok, Wqkv, Wo, W1, W2)

    logits, loss = _head(labels.astype(jnp.int32), x, W_head)
    return logits, loss.reshape(())
